# Initial kernel scaffold; baseline (speedup 1.0000x reference)
import functools

import jax
import jax.numpy as jnp
from jax import lax
from jax.experimental import pallas as pl
from jax.experimental.pallas import tpu as pltpu
from jax.experimental.pallas import tpu_sc as plsc

N = 10000
E = 320000
D = 128
L = 3
G = 64
BN_EPS = 1e-5

NC = 2
NS = 16
NT = NC * NS
EPT = E // NT
CH = 80
NCH = EPT // CH

_sc_mesh = plsc.VectorSubcoreMesh(core_axis_name="c", subcore_axis_name="s")


@functools.partial(
    pl.kernel,
    mesh=_sc_mesh,
    out_type=jax.ShapeDtypeStruct((NC, N, D), jnp.float32),
    scratch_types=[
        pltpu.VMEM((NCH, CH), jnp.int32),
        pltpu.VMEM((NCH, CH), jnp.int32),
        pltpu.VMEM((CH, D), jnp.float32),
        pltpu.VMEM_SHARED((N, D), jnp.float32),
        pltpu.SemaphoreType.DMA,
    ],
)
def _sc_edge_agg(h_hbm, src_hbm, dst_hbm, zero_hbm, out_hbm,
                 src_v, dst_v, rows_v, acc_sh, sem):
    c = lax.axis_index("c")
    s = lax.axis_index("s")
    wid = c * NS + s
    pltpu.sync_copy(src_hbm.at[wid], src_v)
    pltpu.sync_copy(dst_hbm.at[wid], dst_v)
    rps = N // NS
    pltpu.sync_copy(zero_hbm.at[pl.ds(s * rps, rps)],
                    acc_sh.at[pl.ds(s * rps, rps)])
    plsc.subcore_barrier()

    def body(j, carry):
        pltpu.async_copy(h_hbm.at[src_v.at[j]], rows_v, sem).wait()
        pltpu.sync_copy(rows_v, acc_sh.at[dst_v.at[j]], add=True)
        return carry

    lax.fori_loop(0, NCH, body, 0)
    plsc.subcore_barrier()
    pltpu.sync_copy(acc_sh.at[pl.ds(s * rps, rps)],
                    out_hbm.at[c, pl.ds(s * rps, rps)])


def _mlp_body(h_ref, p0_ref, p1_ref, w1_ref, b1_ref, gamma_ref, beta_ref,
              w2_ref, b2_ref, out_ref):
    h = h_ref[...]
    u = h + p0_ref[...] + p1_ref[...]
    t = lax.dot_general(u, w1_ref[...], (((1,), (1,)), ((), ())),
                        preferred_element_type=jnp.float32,
                        precision=lax.Precision.HIGHEST)
    t = t + b1_ref[...]
    mean = jnp.mean(t, axis=0, keepdims=True)
    var = jnp.mean((t - mean) * (t - mean), axis=0, keepdims=True)
    t = (t - mean) * lax.rsqrt(var + BN_EPS) * gamma_ref[...] + beta_ref[...]
    t = jnp.maximum(t, 0.0)
    t = lax.dot_general(t, w2_ref[...], (((1,), (1,)), ((), ())),
                        preferred_element_type=jnp.float32,
                        precision=lax.Precision.HIGHEST)
    t = jnp.maximum(t + b2_ref[...], 0.0)
    out_ref[...] = t + h


def _mlp_pool_body(h_ref, p0_ref, p1_ref, w1_ref, b1_ref, gamma_ref,
                   beta_ref, w2_ref, b2_ref, batch_ref, out_ref, pool_ref):
    _mlp_body(h_ref, p0_ref, p1_ref, w1_ref, b1_ref, gamma_ref, beta_ref,
              w2_ref, b2_ref, out_ref)
    hn = out_ref[...]
    seg = lax.broadcasted_iota(jnp.int32, (N, G), 1)
    onehot = (batch_ref[...] == seg).astype(jnp.float32)
    pool_ref[...] = lax.dot_general(
        onehot, hn, (((0,), (0,)), ((), ())),
        preferred_element_type=jnp.float32,
        precision=lax.Precision.HIGHEST)


def _mlp_call(h, p0, p1, w1, b1, gamma, beta, w2, b2):
    return pl.pallas_call(
        _mlp_body,
        out_shape=jax.ShapeDtypeStruct((N, D), jnp.float32),
    )(h, p0, p1, w1, b1, gamma, beta, w2, b2)


def _mlp_pool_call(h, p0, p1, w1, b1, gamma, beta, w2, b2, batch2):
    return pl.pallas_call(
        _mlp_pool_body,
        out_shape=(jax.ShapeDtypeStruct((N, D), jnp.float32),
                   jax.ShapeDtypeStruct((G, D), jnp.float32)),
    )(h, p0, p1, w1, b1, gamma, beta, w2, b2, batch2)


def kernel(x, edge_index, batch, W1, b1, gamma, beta, W2, b2):
    src3 = edge_index[0].reshape(NT, NCH, CH)
    dst3 = edge_index[1].reshape(NT, NCH, CH)
    zero = jnp.zeros((N, D), jnp.float32)
    batch2 = batch.reshape(N, 1)
    h = x
    pooled = None
    for i in range(L):
        parts = _sc_edge_agg(h, src3, dst3, zero)
        w1 = W1[i]
        b1i = b1[i].reshape(1, D)
        g = gamma[i].reshape(1, D)
        bt = beta[i].reshape(1, D)
        w2 = W2[i]
        b2i = b2[i].reshape(1, D)
        if i < L - 1:
            h = _mlp_call(h, parts[0], parts[1], w1, b1i, g, bt, w2, b2i)
        else:
            h, pooled = _mlp_pool_call(h, parts[0], parts[1], w1, b1i, g, bt,
                                       w2, b2i, batch2)
    return (h, pooled)


# speedup vs baseline: 5.9905x; 5.9905x over previous
import functools

import jax
import jax.numpy as jnp
from jax import lax
from jax.experimental import pallas as pl
from jax.experimental.pallas import tpu as pltpu
from jax.experimental.pallas import tpu_sc as plsc

N = 10000
E = 320000
D = 128
L = 3
G = 64
BN_EPS = 1e-5

NC = 2
NS = 16
NT = NC * NS
EPT = E // NT
CH = 80
NCH = EPT // CH

def _sc_edge_agg_body(h_hbm, src_hbm, dst_hbm, zero_hbm, out_hbm,
                      src_v, dst_v, rows_v, acc_sh, sem):
    c = lax.axis_index("c")
    s = lax.axis_index("s")
    wid = c * NS + s
    pltpu.sync_copy(src_hbm.at[wid], src_v)
    pltpu.sync_copy(dst_hbm.at[wid], dst_v)
    rps = 624
    off = pl.multiple_of(s * rps, 8)
    pltpu.sync_copy(zero_hbm.at[pl.ds(off, rps)], acc_sh.at[pl.ds(off, rps)])

    @pl.when(s == 0)
    def _zero_tail():
        pltpu.sync_copy(zero_hbm.at[pl.ds(NS * rps, N - NS * rps)],
                        acc_sh.at[pl.ds(NS * rps, N - NS * rps)])

    plsc.subcore_barrier()

    def body(j, carry):
        pltpu.async_copy(h_hbm.at[src_v.at[j]], rows_v, sem).wait()
        pltpu.sync_copy(rows_v, acc_sh.at[dst_v.at[j]], add=True)
        return carry

    lax.fori_loop(0, NCH, body, 0)
    plsc.subcore_barrier()
    pltpu.sync_copy(acc_sh.at[pl.ds(off, rps)],
                    out_hbm.at[c, pl.ds(off, rps)])

    @pl.when(s == 0)
    def _out_tail():
        pltpu.sync_copy(acc_sh.at[pl.ds(NS * rps, N - NS * rps)],
                        out_hbm.at[c, pl.ds(NS * rps, N - NS * rps)])


@functools.cache
def _sc_edge_agg():
    mesh = plsc.VectorSubcoreMesh(core_axis_name="c", subcore_axis_name="s",
                                  num_cores=NC, num_subcores=NS)
    return pl.kernel(
        _sc_edge_agg_body,
        mesh=mesh,
        out_type=jax.ShapeDtypeStruct((NC, N, D), jnp.float32),
        scratch_types=[
            pltpu.VMEM((NCH, CH), jnp.int32),
            pltpu.VMEM((NCH, CH), jnp.int32),
            pltpu.VMEM((CH, D), jnp.float32),
            pltpu.VMEM_SHARED((N, D), jnp.float32),
            pltpu.SemaphoreType.DMA,
        ],
    )


def _mlp_body(h_ref, p0_ref, p1_ref, w1_ref, b1_ref, gamma_ref, beta_ref,
              w2_ref, b2_ref, out_ref):
    h = h_ref[...]
    u = h + p0_ref[...] + p1_ref[...]
    t = lax.dot_general(u, w1_ref[...], (((1,), (1,)), ((), ())),
                        preferred_element_type=jnp.float32,
                        precision=lax.Precision.HIGHEST)
    t = t + b1_ref[...]
    mean = jnp.mean(t, axis=0, keepdims=True)
    var = jnp.mean((t - mean) * (t - mean), axis=0, keepdims=True)
    t = (t - mean) * lax.rsqrt(var + BN_EPS) * gamma_ref[...] + beta_ref[...]
    t = jnp.maximum(t, 0.0)
    t = lax.dot_general(t, w2_ref[...], (((1,), (1,)), ((), ())),
                        preferred_element_type=jnp.float32,
                        precision=lax.Precision.HIGHEST)
    t = jnp.maximum(t + b2_ref[...], 0.0)
    out_ref[...] = t + h


def _mlp_pool_body(h_ref, p0_ref, p1_ref, w1_ref, b1_ref, gamma_ref,
                   beta_ref, w2_ref, b2_ref, batch_ref, out_ref, pool_ref):
    _mlp_body(h_ref, p0_ref, p1_ref, w1_ref, b1_ref, gamma_ref, beta_ref,
              w2_ref, b2_ref, out_ref)
    hn = out_ref[...]
    seg = lax.broadcasted_iota(jnp.int32, (N, G), 1)
    onehot = (batch_ref[...] == seg).astype(jnp.float32)
    pool_ref[...] = lax.dot_general(
        onehot, hn, (((0,), (0,)), ((), ())),
        preferred_element_type=jnp.float32,
        precision=lax.Precision.HIGHEST)


def _mlp_call(h, p0, p1, w1, b1, gamma, beta, w2, b2):
    return pl.pallas_call(
        _mlp_body,
        out_shape=jax.ShapeDtypeStruct((N, D), jnp.float32),
    )(h, p0, p1, w1, b1, gamma, beta, w2, b2)


def _mlp_pool_call(h, p0, p1, w1, b1, gamma, beta, w2, b2, batch2):
    return pl.pallas_call(
        _mlp_pool_body,
        out_shape=(jax.ShapeDtypeStruct((N, D), jnp.float32),
                   jax.ShapeDtypeStruct((G, D), jnp.float32)),
    )(h, p0, p1, w1, b1, gamma, beta, w2, b2, batch2)


def kernel(x, edge_index, batch, W1, b1, gamma, beta, W2, b2):
    src3 = edge_index[0].reshape(NT, NCH, CH)
    dst3 = edge_index[1].reshape(NT, NCH, CH)
    zero = jnp.zeros((N, D), jnp.float32)
    batch2 = batch.reshape(N, 1)
    h = x
    pooled = None
    for i in range(L):
        parts = _sc_edge_agg()(h, src3, dst3, zero)
        w1 = W1[i]
        b1i = b1[i].reshape(1, D)
        g = gamma[i].reshape(1, D)
        bt = beta[i].reshape(1, D)
        w2 = W2[i]
        b2i = b2[i].reshape(1, D)
        if i < L - 1:
            h = _mlp_call(h, parts[0], parts[1], w1, b1i, g, bt, w2, b2i)
        else:
            h, pooled = _mlp_pool_call(h, parts[0], parts[1], w1, b1i, g, bt,
                                       w2, b2i, batch2)
    return (h, pooled)
